# baseline (speedup 1.0000x reference)
import functools
import math

import jax
import jax.numpy as jnp
from jax import lax
from jax.experimental import pallas as pl
from jax.experimental.pallas import tpu as pltpu
from jax.experimental.pallas import tpu_sc as plsc

N_NODES = 10000
N_EDGES = 320000
D = 128
DH = D // 2
N_BASIS = 10
RADIAL_NEURONS = 100
NUM_NEIGHBORS = 32

NC = 2
NS = 16
EPT = N_EDGES // NS
B = 80
NB = EPT // B
NPAIR = 5120
RPT = NPAIR // NS
RC = 32
NCHUNK = RPT // RC


def _xl_body(x_ref, w_ref, o_ref):
    o_ref[...] = jnp.dot(x_ref[...], w_ref[...],
                         preferred_element_type=jnp.float32)


def _node_matmul(x, w):
    return pl.pallas_call(
        _xl_body,
        out_shape=jax.ShapeDtypeStruct((N_NODES, D), jnp.float32),
    )(x, w)


BE = 512


def _mlp_body(emb_ref, attr_ref, dst_ref, w0_ref, w1_ref, o_ref, pf_ref):
    h = jnp.dot(emb_ref[...], w0_ref[...], preferred_element_type=jnp.float32)
    h = jnp.maximum(h, 0.0)
    w = jnp.dot(h, w1_ref[...], preferred_element_type=jnp.float32)
    w = w * attr_ref[...]
    o_ref[0, :, :] = w[:, 0:DH]
    o_ref[1, :, :] = w[:, DH:D]
    par = (dst_ref[0, 0, :] & 1).astype(jnp.float32)
    pf_ref[...] = jnp.broadcast_to(par[:, None], (BE, 16))


def _edge_mlp(emb, attr, dst, w0, w1):
    grid = (N_EDGES // BE,)
    return pl.pallas_call(
        _mlp_body,
        grid=grid,
        in_specs=[
            pl.BlockSpec((BE, N_BASIS), lambda i: (i, 0)),
            pl.BlockSpec((BE, 1), lambda i: (i, 0)),
            pl.BlockSpec((1, 1, BE), lambda i: (i, 0, 0)),
            pl.BlockSpec((N_BASIS, RADIAL_NEURONS), lambda i: (0, 0)),
            pl.BlockSpec((RADIAL_NEURONS, D), lambda i: (0, 0)),
        ],
        out_specs=[
            pl.BlockSpec((2, BE, DH), lambda i: (0, i, 0)),
            pl.BlockSpec((BE, 16), lambda i: (i, 0)),
        ],
        out_shape=[
            jax.ShapeDtypeStruct((2, N_EDGES, DH), jnp.float32),
            jax.ShapeDtypeStruct((N_EDGES, 16), jnp.float32),
        ],
    )(emb, attr, dst.reshape(N_EDGES // BE, 1, BE), w0, w1)


def _sc_body(wa_hbm, pf_hbm, src_hbm, dst_hbm, xl_hbm, out_hbm,
             srcv, dstv, wav, pfv, gv, efv, bufv, agg_sh, sem):
    c = lax.axis_index("c")
    s = lax.axis_index("s")

    z16 = jnp.zeros((16,), jnp.float32)

    def zrow(r, acc):
        for j in range(D // 16):
            bufv[r, pl.ds(j * 16, 16)] = z16
        return acc

    lax.fori_loop(0, RC, zrow, 0)
    for h in range(NCHUNK):
        pltpu.sync_copy(bufv, agg_sh.at[pl.ds(s * RPT + h * RC, RC)])
    plsc.subcore_barrier()

    base = s * EPT
    cbase = c * DH
    onef = jnp.ones((16,), jnp.float32)

    def block(b, carry):
        off = base + b * B
        pltpu.sync_copy(src_hbm.at[pl.ds(off, B)], srcv)
        pltpu.sync_copy(dst_hbm.at[pl.ds(off, B)], dstv.at[0])
        pltpu.sync_copy(
            wa_hbm.at[pl.ds((c * N_EDGES + off) * DH, B * DH)], wav)
        pltpu.sync_copy(pf_hbm.at[pl.ds(off * 16, B * 16)], pfv)
        pltpu.async_copy(xl_hbm.at[srcv], gv, sem).wait()

        def mul(e, acc):
            m1 = pfv[pl.ds(e * 16, 16)]
            m0 = onef - m1
            for j in range(DH // 16):
                w = wav[pl.ds(e * DH + j * 16, 16)]
                g = gv[e, pl.ds(cbase + j * 16, 16)]
                p = w * g
                efv[e, pl.ds(j * 16, 16)] = p * m0
                efv[e, pl.ds(DH + j * 16, 16)] = p * m1
            return acc

        lax.fori_loop(0, B, mul, 0)
        for k in range(B // 16):
            sl = pl.ds(k * 16, 16)
            dstv[0, sl] = dstv[0, sl] >> 1
        pltpu.sync_copy(efv, agg_sh.at[dstv.at[0]], add=True)
        return carry

    lax.fori_loop(0, NB, block, 0)
    plsc.subcore_barrier()
    for h in range(NCHUNK):
        pltpu.sync_copy(agg_sh.at[pl.ds(s * RPT + h * RC, RC)], bufv)
        pltpu.sync_copy(
            bufv, out_hbm.at[pl.ds(c * NPAIR + s * RPT + h * RC, RC)])


_sc_mesh = plsc.VectorSubcoreMesh(core_axis_name="c", subcore_axis_name="s")

_sc_scatter = pl.kernel(
    _sc_body,
    out_type=jax.ShapeDtypeStruct((2 * NPAIR, D), jnp.float32),
    mesh=_sc_mesh,
    scratch_types=[
        pltpu.VMEM((B,), jnp.int32),
        pltpu.VMEM((1, B), jnp.int32),
        pltpu.VMEM((B * DH,), jnp.float32),
        pltpu.VMEM((B * 16,), jnp.float32),
        pltpu.VMEM((B, D), jnp.float32),
        pltpu.VMEM((B, D), jnp.float32),
        pltpu.VMEM((RC, D), jnp.float32),
        pltpu.VMEM_SHARED((NPAIR, D), jnp.float32),
        pltpu.SemaphoreType.DMA,
    ],
)


def _final_body(x_ref, a0_ref, a1_ref, wsi_ref, w2_ref, o_ref):
    a = jnp.concatenate([a0_ref[...], a1_ref[...]], axis=1)
    o_ref[...] = (
        jnp.dot(x_ref[...], wsi_ref[...], preferred_element_type=jnp.float32)
        + jnp.dot(a, w2_ref[...], preferred_element_type=jnp.float32)
    )


def _final(x, a0, a1, wsi, w2eff):
    return pl.pallas_call(
        _final_body,
        out_shape=jax.ShapeDtypeStruct((N_NODES, D), jnp.float32),
    )(x, a0, a1, wsi, w2eff)


def kernel(x, edge_src, edge_dst, edge_attr, edge_length_embedded,
           W_si, W_lin1, fc_w0, fc_w1, W_lin2):
    f32 = jnp.float32
    fc0n = (fc_w0 / math.sqrt(N_BASIS)).astype(f32)
    fc1n = (fc_w1 / math.sqrt(RADIAL_NEURONS)).astype(f32)
    wsin = (W_si / math.sqrt(D)).astype(f32)
    wl1n = (W_lin1 / math.sqrt(D)).astype(f32)
    w2eff = (W_lin2 * (0.5 / (math.sqrt(D) * math.sqrt(NUM_NEIGHBORS)))).astype(f32)

    src = edge_src.astype(jnp.int32)
    dst = edge_dst.astype(jnp.int32)

    xl = _node_matmul(x, wl1n)
    wa2, pf = _edge_mlp(edge_length_embedded, edge_attr, dst, fc0n, fc1n)
    agg2 = _sc_scatter(wa2.reshape(2 * N_EDGES * DH), pf.reshape(N_EDGES * 16),
                       src, dst, xl)
    a0 = agg2[0:NPAIR].reshape(2 * NPAIR, DH)[0:N_NODES]
    a1 = agg2[NPAIR:2 * NPAIR].reshape(2 * NPAIR, DH)[0:N_NODES]
    return _final(x, a0, a1, wsin, w2eff)


# speedup vs baseline: 1.3082x; 1.3082x over previous
import functools
import math

import jax
import jax.numpy as jnp
from jax import lax
from jax.experimental import pallas as pl
from jax.experimental.pallas import tpu as pltpu
from jax.experimental.pallas import tpu_sc as plsc

N_NODES = 10000
N_EDGES = 320000
D = 128
DH = D // 2
N_BASIS = 10
RADIAL_NEURONS = 100
NUM_NEIGHBORS = 32

NC = 2
NS = 16
EPT = N_EDGES // NS
B = 80
NB = EPT // B
NPAIR = 5120
RPT = NPAIR // NS
RC = 32
NCHUNK = RPT // RC


def _xl_body(x_ref, w_ref, o_ref):
    o_ref[...] = jnp.dot(x_ref[...], w_ref[...],
                         preferred_element_type=jnp.float32)


def _node_matmul(x, w):
    return pl.pallas_call(
        _xl_body,
        out_shape=jax.ShapeDtypeStruct((N_NODES, D), jnp.float32),
    )(x, w)


BE = 512


def _mlp_body(emb_ref, attr_ref, dst_ref, w0_ref, w1_ref, o_ref, pf_ref):
    h = jnp.dot(emb_ref[...], w0_ref[...], preferred_element_type=jnp.float32)
    h = jnp.maximum(h, 0.0)
    w = jnp.dot(h, w1_ref[...], preferred_element_type=jnp.float32)
    w = w * attr_ref[...]
    o_ref[0, :, :] = w[:, 0:DH]
    o_ref[1, :, :] = w[:, DH:D]
    par = (dst_ref[0, 0, :] & 1).astype(jnp.float32)
    pf_ref[...] = jnp.broadcast_to(par[:, None], (BE, 16))


def _edge_mlp(emb, attr, dst, w0, w1):
    grid = (N_EDGES // BE,)
    return pl.pallas_call(
        _mlp_body,
        grid=grid,
        in_specs=[
            pl.BlockSpec((BE, N_BASIS), lambda i: (i, 0)),
            pl.BlockSpec((BE, 1), lambda i: (i, 0)),
            pl.BlockSpec((1, 1, BE), lambda i: (i, 0, 0)),
            pl.BlockSpec((N_BASIS, RADIAL_NEURONS), lambda i: (0, 0)),
            pl.BlockSpec((RADIAL_NEURONS, D), lambda i: (0, 0)),
        ],
        out_specs=[
            pl.BlockSpec((2, BE, DH), lambda i: (0, i, 0)),
            pl.BlockSpec((BE, 16), lambda i: (i, 0)),
        ],
        out_shape=[
            jax.ShapeDtypeStruct((2, N_EDGES, DH), jnp.float32),
            jax.ShapeDtypeStruct((N_EDGES, 16), jnp.float32),
        ],
    )(emb, attr, dst.reshape(N_EDGES // BE, 1, BE), w0, w1)


def _sc_body(wa_hbm, pf_hbm, src_hbm, dst_hbm, xl_hbm, out_hbm,
             srcv0, srcv1, dstv0, dstv1, wav0, wav1, pfv0, pfv1,
             gv0, gv1, efv0, efv1, bufv, agg_sh,
             sin0, sin1, sg0, sg1, ssc0, ssc1):
    c = lax.axis_index("c")
    s = lax.axis_index("s")

    z16 = jnp.zeros((16,), jnp.float32)

    def zrow(r, acc):
        for j in range(D // 16):
            bufv[r, pl.ds(j * 16, 16)] = z16
        return acc

    lax.fori_loop(0, RC, zrow, 0)
    for h in range(NCHUNK):
        pltpu.sync_copy(bufv, agg_sh.at[pl.ds(s * RPT + h * RC, RC)])
    plsc.subcore_barrier()

    base = s * EPT
    cbase = c * DH
    onef = jnp.ones((16,), jnp.float32)

    srcs = (srcv0, srcv1)
    dsts = (dstv0, dstv1)
    wavs = (wav0, wav1)
    pfvs = (pfv0, pfv1)
    gvs = (gv0, gv1)
    efvs = (efv0, efv1)
    sins = (sin0, sin1)
    sgs = (sg0, sg1)
    sscs = (ssc0, ssc1)

    def in_copies(b, p):
        off = base + b * B
        return (
            pltpu.make_async_copy(
                src_hbm.at[pl.ds(off, B)], srcs[p], sins[p]),
            pltpu.make_async_copy(
                wa_hbm.at[pl.ds((c * N_EDGES + off) * DH, B * DH)],
                wavs[p], sins[p]),
            pltpu.make_async_copy(
                pf_hbm.at[pl.ds(off * 16, B * 16)], pfvs[p], sins[p]),
        )

    def scat(p):
        return pltpu.make_async_copy(
            efvs[p], agg_sh.at[dsts[p].at[0]], sscs[p])

    for p in range(2):
        for cp in in_copies(p, p):
            cp.start()

    def gloop(g, carry):
        for p in range(2):
            b = 2 * g + p
            off = base + b * B
            for cp in in_copies(b, p):
                cp.wait()
            pltpu.make_async_copy(xl_hbm.at[srcs[p]], gvs[p], sgs[p]).start()

            @pl.when(g > 0)
            def _():
                scat(p).wait()

            pltpu.sync_copy(dst_hbm.at[pl.ds(off, B)], dsts[p].at[0])
            for k in range(B // 16):
                sl = pl.ds(k * 16, 16)
                dsts[p][0, sl] = dsts[p][0, sl] >> 1

            pltpu.make_async_copy(xl_hbm.at[srcs[p]], gvs[p], sgs[p]).wait()

            def mul(e, acc):
                m1 = pfvs[p][pl.ds(e * 16, 16)]
                m0 = onef - m1
                for j in range(DH // 16):
                    w = wavs[p][pl.ds(e * DH + j * 16, 16)]
                    gg = gvs[p][e, pl.ds(cbase + j * 16, 16)]
                    pr = w * gg
                    efvs[p][e, pl.ds(j * 16, 16)] = pr * m0
                    efvs[p][e, pl.ds(DH + j * 16, 16)] = pr * m1
                return acc

            lax.fori_loop(0, B, mul, 0)
            scat(p).start(add=True)

            @pl.when(b + 2 < NB)
            def _():
                for cp in in_copies(b + 2, p):
                    cp.start()
        return carry

    lax.fori_loop(0, NB // 2, gloop, 0)
    for p in range(2):
        scat(p).wait()
    plsc.subcore_barrier()
    for h in range(NCHUNK):
        pltpu.sync_copy(agg_sh.at[pl.ds(s * RPT + h * RC, RC)], bufv)
        pltpu.sync_copy(
            bufv, out_hbm.at[pl.ds(c * NPAIR + s * RPT + h * RC, RC)])


_sc_mesh = plsc.VectorSubcoreMesh(core_axis_name="c", subcore_axis_name="s")

_sc_scatter = pl.kernel(
    _sc_body,
    out_type=jax.ShapeDtypeStruct((2 * NPAIR, D), jnp.float32),
    mesh=_sc_mesh,
    scratch_types=[
        pltpu.VMEM((B,), jnp.int32),
        pltpu.VMEM((B,), jnp.int32),
        pltpu.VMEM((1, B), jnp.int32),
        pltpu.VMEM((1, B), jnp.int32),
        pltpu.VMEM((B * DH,), jnp.float32),
        pltpu.VMEM((B * DH,), jnp.float32),
        pltpu.VMEM((B * 16,), jnp.float32),
        pltpu.VMEM((B * 16,), jnp.float32),
        pltpu.VMEM((B, D), jnp.float32),
        pltpu.VMEM((B, D), jnp.float32),
        pltpu.VMEM((B, D), jnp.float32),
        pltpu.VMEM((B, D), jnp.float32),
        pltpu.VMEM((RC, D), jnp.float32),
        pltpu.VMEM_SHARED((NPAIR, D), jnp.float32),
        pltpu.SemaphoreType.DMA,
        pltpu.SemaphoreType.DMA,
        pltpu.SemaphoreType.DMA,
        pltpu.SemaphoreType.DMA,
        pltpu.SemaphoreType.DMA,
        pltpu.SemaphoreType.DMA,
    ],
)


def _final_body(x_ref, a0_ref, a1_ref, wsi_ref, w2_ref, o_ref):
    a = jnp.concatenate([a0_ref[...], a1_ref[...]], axis=1)
    o_ref[...] = (
        jnp.dot(x_ref[...], wsi_ref[...], preferred_element_type=jnp.float32)
        + jnp.dot(a, w2_ref[...], preferred_element_type=jnp.float32)
    )


def _final(x, a0, a1, wsi, w2eff):
    return pl.pallas_call(
        _final_body,
        out_shape=jax.ShapeDtypeStruct((N_NODES, D), jnp.float32),
    )(x, a0, a1, wsi, w2eff)


def kernel(x, edge_src, edge_dst, edge_attr, edge_length_embedded,
           W_si, W_lin1, fc_w0, fc_w1, W_lin2):
    f32 = jnp.float32
    fc0n = (fc_w0 / math.sqrt(N_BASIS)).astype(f32)
    fc1n = (fc_w1 / math.sqrt(RADIAL_NEURONS)).astype(f32)
    wsin = (W_si / math.sqrt(D)).astype(f32)
    wl1n = (W_lin1 / math.sqrt(D)).astype(f32)
    w2eff = (W_lin2 * (0.5 / (math.sqrt(D) * math.sqrt(NUM_NEIGHBORS)))).astype(f32)

    src = edge_src.astype(jnp.int32)
    dst = edge_dst.astype(jnp.int32)

    xl = _node_matmul(x, wl1n)
    wa2, pf = _edge_mlp(edge_length_embedded, edge_attr, dst, fc0n, fc1n)
    agg2 = _sc_scatter(wa2.reshape(2 * N_EDGES * DH), pf.reshape(N_EDGES * 16),
                       src, dst, xl)
    a0 = agg2[0:NPAIR].reshape(2 * NPAIR, DH)[0:N_NODES]
    a1 = agg2[NPAIR:2 * NPAIR].reshape(2 * NPAIR, DH)[0:N_NODES]
    return _final(x, a0, a1, wsin, w2eff)


# speedup vs baseline: 1.7839x; 1.3636x over previous
import functools
import math

import jax
import jax.numpy as jnp
from jax import lax
from jax.experimental import pallas as pl
from jax.experimental.pallas import tpu as pltpu
from jax.experimental.pallas import tpu_sc as plsc

N_NODES = 10000
N_EDGES = 320000
D = 128
DH = D // 2
N_BASIS = 10
RADIAL_NEURONS = 100
NUM_NEIGHBORS = 32

NC = 2
NS = 16
EPT = N_EDGES // NS
B = 80
NB = EPT // B
NPAIR = 5120
RPT = NPAIR // NS
RC = 32
NCHUNK = RPT // RC


def _xl_body(x_ref, w_ref, o_ref):
    o_ref[...] = jnp.dot(x_ref[...], w_ref[...],
                         preferred_element_type=jnp.float32)


def _node_matmul(x, w):
    return pl.pallas_call(
        _xl_body,
        out_shape=jax.ShapeDtypeStruct((N_NODES, D), jnp.float32),
    )(x, w)


BE = 3200


def _mlp_body(emb_ref, attr_ref, dst_ref, w0_ref, w1_ref, o_ref, pf_ref):
    h = jnp.dot(emb_ref[...], w0_ref[...], preferred_element_type=jnp.float32)
    h = jnp.maximum(h, 0.0)
    w = jnp.dot(h, w1_ref[...], preferred_element_type=jnp.float32)
    o_ref[...] = w * attr_ref[...]
    par = (dst_ref[0, 0, :] & 1).astype(jnp.float32)
    pf_ref[...] = jnp.broadcast_to(par[:, None], (BE, 16))


def _edge_mlp(emb, attr, dst, w0, w1):
    grid = (N_EDGES // BE,)
    return pl.pallas_call(
        _mlp_body,
        grid=grid,
        in_specs=[
            pl.BlockSpec((BE, N_BASIS), lambda i: (i, 0)),
            pl.BlockSpec((BE, 1), lambda i: (i, 0)),
            pl.BlockSpec((1, 1, BE), lambda i: (i, 0, 0)),
            pl.BlockSpec((N_BASIS, RADIAL_NEURONS), lambda i: (0, 0)),
            pl.BlockSpec((RADIAL_NEURONS, D), lambda i: (0, 0)),
        ],
        out_specs=[
            pl.BlockSpec((BE, D), lambda i: (i, 0)),
            pl.BlockSpec((BE, 16), lambda i: (i, 0)),
        ],
        out_shape=[
            jax.ShapeDtypeStruct((N_EDGES, D), jnp.float32),
            jax.ShapeDtypeStruct((N_EDGES, 16), jnp.float32),
        ],
    )(emb, attr, dst.reshape(N_EDGES // BE, 1, BE), w0, w1)


def _sc_body(wa_hbm, pf_hbm, src_hbm, dst_hbm, xl_hbm, out_hbm,
             srcv0, srcv1, dstv0, dstv1, wav0, wav1, pfv0, pfv1,
             gv0, gv1, efv0, efv1, bufv, agg_sh,
             sin0, sin1, sg0, sg1, ssc0, ssc1):
    c = lax.axis_index("c")
    s = lax.axis_index("s")

    z16 = jnp.zeros((16,), jnp.float32)

    def zrow(r, acc):
        for j in range(D // 16):
            bufv[r, pl.ds(j * 16, 16)] = z16
        return acc

    lax.fori_loop(0, RC, zrow, 0)
    for h in range(NCHUNK):
        pltpu.sync_copy(bufv, agg_sh.at[pl.ds(s * RPT + h * RC, RC)])
    plsc.subcore_barrier()

    base = s * EPT
    cbase = c * DH
    onef = jnp.ones((16,), jnp.float32)

    srcs = (srcv0, srcv1)
    dsts = (dstv0, dstv1)
    wavs = (wav0, wav1)
    pfvs = (pfv0, pfv1)
    gvs = (gv0, gv1)
    efvs = (efv0, efv1)
    sins = (sin0, sin1)
    sgs = (sg0, sg1)
    sscs = (ssc0, ssc1)

    def in_copies(b, p):
        off = base + b * B
        return (
            pltpu.make_async_copy(
                src_hbm.at[pl.ds(off, B)], srcs[p], sins[p]),
            pltpu.make_async_copy(
                wa_hbm.at[pl.ds(off * D, B * D)], wavs[p], sins[p]),
            pltpu.make_async_copy(
                pf_hbm.at[pl.ds(off * 16, B * 16)], pfvs[p], sins[p]),
        )

    def scat(p):
        return pltpu.make_async_copy(
            efvs[p], agg_sh.at[dsts[p].at[0]], sscs[p])

    for p in range(2):
        for cp in in_copies(p, p):
            cp.start()

    def gloop(g, carry):
        for p in range(2):
            b = 2 * g + p
            off = base + b * B
            for cp in in_copies(b, p):
                cp.wait()
            pltpu.make_async_copy(xl_hbm.at[srcs[p]], gvs[p], sgs[p]).start()

            @pl.when(g > 0)
            def _():
                scat(p).wait()

            pltpu.sync_copy(dst_hbm.at[pl.ds(off, B)], dsts[p].at[0])
            for k in range(B // 16):
                sl = pl.ds(k * 16, 16)
                dsts[p][0, sl] = dsts[p][0, sl] >> 1

            pltpu.make_async_copy(xl_hbm.at[srcs[p]], gvs[p], sgs[p]).wait()

            def mul(e, acc):
                m1 = pfvs[p][pl.ds(e * 16, 16)]
                m0 = onef - m1
                for j in range(DH // 16):
                    w = wavs[p][pl.ds(e * D + cbase + j * 16, 16)]
                    gg = gvs[p][e, pl.ds(cbase + j * 16, 16)]
                    pr = w * gg
                    efvs[p][e, pl.ds(j * 16, 16)] = pr * m0
                    efvs[p][e, pl.ds(DH + j * 16, 16)] = pr * m1
                return acc

            lax.fori_loop(0, B, mul, 0)
            scat(p).start(add=True)

            @pl.when(b + 2 < NB)
            def _():
                for cp in in_copies(b + 2, p):
                    cp.start()
        return carry

    lax.fori_loop(0, NB // 2, gloop, 0)
    for p in range(2):
        scat(p).wait()
    plsc.subcore_barrier()
    for h in range(NCHUNK):
        pltpu.sync_copy(agg_sh.at[pl.ds(s * RPT + h * RC, RC)], bufv)
        pltpu.sync_copy(
            bufv, out_hbm.at[pl.ds(c * NPAIR + s * RPT + h * RC, RC)])


_sc_mesh = plsc.VectorSubcoreMesh(core_axis_name="c", subcore_axis_name="s")

_sc_scatter = pl.kernel(
    _sc_body,
    out_type=jax.ShapeDtypeStruct((2 * NPAIR, D), jnp.float32),
    mesh=_sc_mesh,
    scratch_types=[
        pltpu.VMEM((B,), jnp.int32),
        pltpu.VMEM((B,), jnp.int32),
        pltpu.VMEM((1, B), jnp.int32),
        pltpu.VMEM((1, B), jnp.int32),
        pltpu.VMEM((B * D,), jnp.float32),
        pltpu.VMEM((B * D,), jnp.float32),
        pltpu.VMEM((B * 16,), jnp.float32),
        pltpu.VMEM((B * 16,), jnp.float32),
        pltpu.VMEM((B, D), jnp.float32),
        pltpu.VMEM((B, D), jnp.float32),
        pltpu.VMEM((B, D), jnp.float32),
        pltpu.VMEM((B, D), jnp.float32),
        pltpu.VMEM((RC, D), jnp.float32),
        pltpu.VMEM_SHARED((NPAIR, D), jnp.float32),
        pltpu.SemaphoreType.DMA,
        pltpu.SemaphoreType.DMA,
        pltpu.SemaphoreType.DMA,
        pltpu.SemaphoreType.DMA,
        pltpu.SemaphoreType.DMA,
        pltpu.SemaphoreType.DMA,
    ],
)


def _final_body(x_ref, a0_ref, a1_ref, wsi_ref, w2_ref, o_ref):
    a = jnp.concatenate([a0_ref[...], a1_ref[...]], axis=1)
    o_ref[...] = (
        jnp.dot(x_ref[...], wsi_ref[...], preferred_element_type=jnp.float32)
        + jnp.dot(a, w2_ref[...], preferred_element_type=jnp.float32)
    )


def _final(x, a0, a1, wsi, w2eff):
    return pl.pallas_call(
        _final_body,
        out_shape=jax.ShapeDtypeStruct((N_NODES, D), jnp.float32),
    )(x, a0, a1, wsi, w2eff)


def kernel(x, edge_src, edge_dst, edge_attr, edge_length_embedded,
           W_si, W_lin1, fc_w0, fc_w1, W_lin2):
    f32 = jnp.float32
    fc0n = (fc_w0 / math.sqrt(N_BASIS)).astype(f32)
    fc1n = (fc_w1 / math.sqrt(RADIAL_NEURONS)).astype(f32)
    wsin = (W_si / math.sqrt(D)).astype(f32)
    wl1n = (W_lin1 / math.sqrt(D)).astype(f32)
    w2eff = (W_lin2 * (0.5 / (math.sqrt(D) * math.sqrt(NUM_NEIGHBORS)))).astype(f32)

    src = edge_src.astype(jnp.int32)
    dst = edge_dst.astype(jnp.int32)

    xl = _node_matmul(x, wl1n)
    wa2, pf = _edge_mlp(edge_length_embedded, edge_attr, dst, fc0n, fc1n)
    agg2 = _sc_scatter(wa2.reshape(N_EDGES * D), pf.reshape(N_EDGES * 16),
                       src, dst, xl)
    a0 = agg2[0:NPAIR].reshape(2 * NPAIR, DH)[0:N_NODES]
    a1 = agg2[NPAIR:2 * NPAIR].reshape(2 * NPAIR, DH)[0:N_NODES]
    return _final(x, a0, a1, wsin, w2eff)
